# Initial kernel scaffold; baseline (speedup 1.0000x reference)
import jax
import jax.numpy as jnp
from jax.experimental import pallas as pl


def kernel(node_features, edge_features, global_features, params, senders, receivers, is_trainning, prebuild_graph):
    raise NotImplementedError("write your pallas kernel here")


# speedup vs baseline: 2.3661x; 2.3661x over previous
import functools

import jax
import jax.numpy as jnp
from jax import lax
from jax.experimental import pallas as pl
from jax.experimental.pallas import tpu as pltpu
from jax.experimental.pallas import tpu_sc as plsc

N_NODES = 10000
N_EDGES = 160000
LATENT = 128
STEPS = 15
OUT_DIM = 3

EBLK = 640
NBLK = 1000

NC = 2
NS = 16
NW = NC * NS
CH = 128
NCHUNK = N_EDGES // CH
CPW = -(-NCHUNK // NW)
NPAD = 10240
NSTRIPE = NPAD // NS

_f32 = jnp.float32


def _ln(d, s, b):
    m = jnp.mean(d, axis=-1, keepdims=True)
    v = jnp.mean(jnp.square(d - m), axis=-1, keepdims=True)
    return (d - m) * lax.rsqrt(v + 1e-5) * s + b


def _dot(a, b):
    return jnp.dot(a, b, preferred_element_type=jnp.float32)


def _enc_body(x_ref, w1_ref, b1_ref, w2_ref, b2_ref, w3_ref, b3_ref,
              lns_ref, lnb_ref, out_ref):
    h = jnp.maximum(_dot(x_ref[...], w1_ref[...]) + b1_ref[...], 0.0)
    h = jnp.maximum(_dot(h, w2_ref[...]) + b2_ref[...], 0.0)
    d = _dot(h, w3_ref[...]) + b3_ref[...]
    out_ref[...] = _ln(d, lns_ref[...], lnb_ref[...])


def _encoder(x, w1, b1, w2, b2, w3, b3, lns, lnb, blk):
    n, f = x.shape
    grid = n // blk
    const = lambda i: (0, 0)
    return pl.pallas_call(
        _enc_body,
        grid=(grid,),
        in_specs=[
            pl.BlockSpec((blk, f), lambda i: (i, 0)),
            pl.BlockSpec((f, LATENT), const),
            pl.BlockSpec((1, LATENT), const),
            pl.BlockSpec((LATENT, LATENT), const),
            pl.BlockSpec((1, LATENT), const),
            pl.BlockSpec((LATENT, LATENT), const),
            pl.BlockSpec((1, LATENT), const),
            pl.BlockSpec((1, LATENT), const),
            pl.BlockSpec((1, LATENT), const),
        ],
        out_specs=pl.BlockSpec((blk, LATENT), lambda i: (i, 0)),
        out_shape=jax.ShapeDtypeStruct((n, LATENT), _f32),
    )(x, w1, b1.reshape(1, -1), w2, b2.reshape(1, -1), w3, b3.reshape(1, -1),
      lns.reshape(1, -1), lnb.reshape(1, -1))


def _pq_body(x_ref, ws_ref, wr_ref, p_ref, q_ref):
    x = x_ref[...]
    p_ref[...] = _dot(x, ws_ref[...])
    q_ref[...] = _dot(x, wr_ref[...])


def _pq(node, ws, wr):
    const = lambda i: (0, 0)
    return pl.pallas_call(
        _pq_body,
        grid=(N_NODES // NBLK,),
        in_specs=[
            pl.BlockSpec((NBLK, LATENT), lambda i: (i, 0)),
            pl.BlockSpec((LATENT, LATENT), const),
            pl.BlockSpec((LATENT, LATENT), const),
        ],
        out_specs=[
            pl.BlockSpec((NBLK, LATENT), lambda i: (i, 0)),
            pl.BlockSpec((NBLK, LATENT), lambda i: (i, 0)),
        ],
        out_shape=[
            jax.ShapeDtypeStruct((N_NODES, LATENT), _f32),
            jax.ShapeDtypeStruct((N_NODES, LATENT), _f32),
        ],
    )(node, ws, wr)


def _edge_body(e_ref, ps_ref, qr_ref, w1_ref, b1_ref, w2_ref, b2_ref,
               w3_ref, b3_ref, lns_ref, lnb_ref, out_ref, sum_ref):
    x = e_ref[...]
    h = _dot(x, w1_ref[...]) + ps_ref[...] + qr_ref[...] + b1_ref[...]
    h = jnp.maximum(h, 0.0)
    h = jnp.maximum(_dot(h, w2_ref[...]) + b2_ref[...], 0.0)
    d = _dot(h, w3_ref[...]) + b3_ref[...]
    o = x + _ln(d, lns_ref[...], lnb_ref[...])
    out_ref[...] = o

    @pl.when(pl.program_id(0) == 0)
    def _():
        sum_ref[...] = jnp.zeros_like(sum_ref)

    sum_ref[...] += jnp.sum(o, axis=0, keepdims=True)


def _edge_step(edge, ps, qr, w1, b1, w2, b2, w3, b3, lns, lnb):
    const = lambda i: (0, 0)
    row = lambda i: (i, 0)
    return pl.pallas_call(
        _edge_body,
        grid=(N_EDGES // EBLK,),
        in_specs=[
            pl.BlockSpec((EBLK, LATENT), row),
            pl.BlockSpec((EBLK, LATENT), row),
            pl.BlockSpec((EBLK, LATENT), row),
            pl.BlockSpec((LATENT, LATENT), const),
            pl.BlockSpec((1, LATENT), const),
            pl.BlockSpec((LATENT, LATENT), const),
            pl.BlockSpec((1, LATENT), const),
            pl.BlockSpec((LATENT, LATENT), const),
            pl.BlockSpec((1, LATENT), const),
            pl.BlockSpec((1, LATENT), const),
            pl.BlockSpec((1, LATENT), const),
        ],
        out_specs=[
            pl.BlockSpec((EBLK, LATENT), row),
            pl.BlockSpec((1, LATENT), const),
        ],
        out_shape=[
            jax.ShapeDtypeStruct((N_EDGES, LATENT), _f32),
            jax.ShapeDtypeStruct((1, LATENT), _f32),
        ],
    )(edge, ps, qr, w1, b1.reshape(1, -1), w2, b2.reshape(1, -1), w3,
      b3.reshape(1, -1), lns.reshape(1, -1), lnb.reshape(1, -1))


def _node_body(n_ref, a0_ref, a1_ref, g_ref, wn_ref, wa_ref, wg_ref, b1_ref,
               w2_ref, b2_ref, w3_ref, b3_ref, lns_ref, lnb_ref,
               out_ref, sum_ref):
    x = n_ref[...]
    agg = a0_ref[...] + a1_ref[...]
    h = (_dot(x, wn_ref[...]) + _dot(agg, wa_ref[...])
         + _dot(g_ref[...], wg_ref[...]) + b1_ref[...])
    h = jnp.maximum(h, 0.0)
    h = jnp.maximum(_dot(h, w2_ref[...]) + b2_ref[...], 0.0)
    d = _dot(h, w3_ref[...]) + b3_ref[...]
    o = x + _ln(d, lns_ref[...], lnb_ref[...])
    out_ref[...] = o

    @pl.when(pl.program_id(0) == 0)
    def _():
        sum_ref[...] = jnp.zeros_like(sum_ref)

    sum_ref[...] += jnp.sum(o, axis=0, keepdims=True)


def _node_step(node, agg0, agg1, g, wn, wa, wg, b1, w2, b2, w3, b3, lns, lnb):
    const = lambda i: (0, 0)
    row = lambda i: (i, 0)
    return pl.pallas_call(
        _node_body,
        grid=(N_NODES // NBLK,),
        in_specs=[
            pl.BlockSpec((NBLK, LATENT), row),
            pl.BlockSpec((NBLK, LATENT), row),
            pl.BlockSpec((NBLK, LATENT), row),
            pl.BlockSpec((1, LATENT), const),
            pl.BlockSpec((LATENT, LATENT), const),
            pl.BlockSpec((LATENT, LATENT), const),
            pl.BlockSpec((LATENT, LATENT), const),
            pl.BlockSpec((1, LATENT), const),
            pl.BlockSpec((LATENT, LATENT), const),
            pl.BlockSpec((1, LATENT), const),
            pl.BlockSpec((LATENT, LATENT), const),
            pl.BlockSpec((1, LATENT), const),
            pl.BlockSpec((1, LATENT), const),
            pl.BlockSpec((1, LATENT), const),
        ],
        out_specs=[
            pl.BlockSpec((NBLK, LATENT), row),
            pl.BlockSpec((1, LATENT), const),
        ],
        out_shape=[
            jax.ShapeDtypeStruct((N_NODES, LATENT), _f32),
            jax.ShapeDtypeStruct((1, LATENT), _f32),
        ],
    )(node, agg0, agg1, g, wn, wa, wg, b1.reshape(1, -1), w2,
      b2.reshape(1, -1), w3, b3.reshape(1, -1), lns.reshape(1, -1),
      lnb.reshape(1, -1))


def _global_body(g_ref, ns_ref, es_ref, wg_ref, wn_ref, we_ref, b1_ref,
                 w2_ref, b2_ref, w3_ref, b3_ref, lns_ref, lnb_ref, out_ref):
    g = g_ref[...]
    h = (_dot(g, wg_ref[...])
         + _dot(ns_ref[...] * (1.0 / N_NODES), wn_ref[...])
         + _dot(es_ref[...] * (1.0 / N_EDGES), we_ref[...])
         + b1_ref[...])
    h = jnp.maximum(h, 0.0)
    h = jnp.maximum(_dot(h, w2_ref[...]) + b2_ref[...], 0.0)
    d = _dot(h, w3_ref[...]) + b3_ref[...]
    out_ref[...] = g + _ln(d, lns_ref[...], lnb_ref[...])


def _global_step(g, nsum, esum, wg, wn, we, b1, w2, b2, w3, b3, lns, lnb):
    return pl.pallas_call(
        _global_body,
        out_shape=jax.ShapeDtypeStruct((1, LATENT), _f32),
    )(g, nsum, esum, wg, wn, we, b1.reshape(1, -1), w2, b2.reshape(1, -1),
      w3, b3.reshape(1, -1), lns.reshape(1, -1), lnb.reshape(1, -1))


def _dec_body(x_ref, w1_ref, b1_ref, w2_ref, b2_ref, w3_ref, b3_ref, out_ref):
    h = jnp.maximum(_dot(x_ref[...], w1_ref[...]) + b1_ref[...], 0.0)
    h = jnp.maximum(_dot(h, w2_ref[...]) + b2_ref[...], 0.0)
    out_ref[...] = _dot(h, w3_ref[...]) + b3_ref[...]


def _decoder(node, w1, b1, w2, b2, w3, b3):
    const = lambda i: (0, 0)
    w3p = jnp.pad(w3, ((0, 0), (0, LATENT - OUT_DIM)))
    b3p = jnp.pad(b3, (0, LATENT - OUT_DIM))
    out = pl.pallas_call(
        _dec_body,
        grid=(N_NODES // NBLK,),
        in_specs=[
            pl.BlockSpec((NBLK, LATENT), lambda i: (i, 0)),
            pl.BlockSpec((LATENT, LATENT), const),
            pl.BlockSpec((1, LATENT), const),
            pl.BlockSpec((LATENT, LATENT), const),
            pl.BlockSpec((1, LATENT), const),
            pl.BlockSpec((LATENT, LATENT), const),
            pl.BlockSpec((1, LATENT), const),
        ],
        out_specs=pl.BlockSpec((NBLK, LATENT), lambda i: (i, 0)),
        out_shape=jax.ShapeDtypeStruct((N_NODES, LATENT), _f32),
    )(node, w1, b1.reshape(1, -1), w2, b2.reshape(1, -1), w3p,
      b3p.reshape(1, -1))
    return out[:, :OUT_DIM]


@functools.cache
def _build_sc_gather2():
    mesh = plsc.VectorSubcoreMesh(core_axis_name="c", subcore_axis_name="s",
                                  num_cores=NC, num_subcores=NS)

    @functools.partial(
        pl.kernel,
        out_type=(
            jax.ShapeDtypeStruct((N_EDGES, LATENT), _f32),
            jax.ShapeDtypeStruct((N_EDGES, LATENT), _f32),
        ),
        mesh=mesh,
        scratch_types=[
            pltpu.VMEM((CH,), jnp.int32),
            pltpu.VMEM((CH,), jnp.int32),
            pltpu.VMEM((CH, LATENT), _f32),
            pltpu.VMEM((CH, LATENT), _f32),
            pltpu.SemaphoreType.DMA,
            pltpu.SemaphoreType.DMA,
        ],
    )
    def sc_gather2(p_hbm, q_hbm, snd_hbm, rcv_hbm, ps_out, qr_out,
                   sidx, ridx, pbuf, qbuf, sem1, sem2):
        wid = lax.axis_index("s") * NC + lax.axis_index("c")

        def body(i, carry):
            c = wid + i * NW

            @pl.when(c < NCHUNK)
            def _():
                base = c * CH
                pltpu.sync_copy(snd_hbm.at[pl.ds(base, CH)], sidx)
                pltpu.sync_copy(rcv_hbm.at[pl.ds(base, CH)], ridx)
                cp1 = pltpu.async_copy(p_hbm.at[sidx], pbuf, sem1)
                cp2 = pltpu.async_copy(q_hbm.at[ridx], qbuf, sem2)
                cp1.wait()
                cp2.wait()
                pltpu.sync_copy(pbuf, ps_out.at[pl.ds(base, CH)])
                pltpu.sync_copy(qbuf, qr_out.at[pl.ds(base, CH)])

            return carry

        lax.fori_loop(0, CPW, body, 0)

    return sc_gather2


def _sc_gather2(p_tab, q_tab, snd, rcv):
    return _build_sc_gather2()(p_tab, q_tab, snd, rcv)


@functools.cache
def _build_sc_scatter():
    mesh = plsc.VectorSubcoreMesh(core_axis_name="c", subcore_axis_name="s",
                                  num_cores=NC, num_subcores=NS)

    @functools.partial(
        pl.kernel,
        out_type=jax.ShapeDtypeStruct((NC, NPAD, LATENT), _f32),
        mesh=mesh,
        scratch_types=[
            pltpu.VMEM((CH, LATENT), _f32),
            pltpu.VMEM((CH,), jnp.int32),
            pltpu.VMEM((64, LATENT), _f32),
            pltpu.VMEM_SHARED((NPAD, LATENT), _f32),
        ],
    )
    def sc_scatter(edge_hbm, rcv_hbm, out_hbm, ebuf, ridx, stage, acc):
        cid = lax.axis_index("c")
        sid = lax.axis_index("s")
        wid = sid * NC + cid

        def zrow(i, carry):
            for k in range(LATENT // 16):
                stage[i, pl.ds(k * 16, 16)] = jnp.zeros((16,), _f32)
            return carry

        lax.fori_loop(0, 64, zrow, 0)

        def zcopy(j, carry):
            pltpu.sync_copy(stage, acc.at[pl.ds(sid * NSTRIPE + j * 64, 64)])
            return carry

        lax.fori_loop(0, NSTRIPE // 64, zcopy, 0)
        plsc.subcore_barrier()

        def body(i, carry):
            c = wid + i * NW

            @pl.when(c < NCHUNK)
            def _():
                base = c * CH
                pltpu.sync_copy(rcv_hbm.at[pl.ds(base, CH)], ridx)
                pltpu.sync_copy(edge_hbm.at[pl.ds(base, CH)], ebuf)
                pltpu.sync_copy(ebuf, acc.at[ridx], add=True)

            return carry

        lax.fori_loop(0, CPW, body, 0)
        plsc.subcore_barrier()
        pltpu.sync_copy(acc.at[pl.ds(sid * NSTRIPE, NSTRIPE)],
                        out_hbm.at[cid, pl.ds(sid * NSTRIPE, NSTRIPE)])

    return sc_scatter


def _sc_scatter(edge, rcv):
    return _build_sc_scatter()(edge, rcv)


def kernel(node_features, edge_features, global_features, params, senders,
           receivers, is_trainning, prebuild_graph):
    p = params

    nstd = jnp.maximum(p['node_norm_std'], 1e-8)
    w1n = p['enc_n_w1'] / nstd[:, None]
    b1n = p['enc_n_b1'] - _foldb(p['node_norm_mean'], nstd, p['enc_n_w1'])
    estd = jnp.maximum(p['edge_norm_std'], 1e-8)
    w1e = p['enc_e_w1'] / estd[:, None]
    b1e = p['enc_e_b1'] - _foldb(p['edge_norm_mean'], estd, p['enc_e_w1'])

    xn = jnp.pad(node_features, ((0, 0), (0, 4)))
    w1n = jnp.pad(w1n, ((0, 4), (0, 0)))
    xe = jnp.pad(edge_features, ((0, 0), (0, 2)))
    w1e = jnp.pad(w1e, ((0, 2), (0, 0)))

    node = _encoder(xn, w1n, b1n, p['enc_n_w2'], p['enc_n_b2'],
                    p['enc_n_w3'], p['enc_n_b3'], p['enc_n_ln_s'],
                    p['enc_n_ln_b'], NBLK)
    edge = _encoder(xe, w1e, b1e, p['enc_e_w2'], p['enc_e_b2'],
                    p['enc_e_w3'], p['enc_e_b3'], p['enc_e_ln_s'],
                    p['enc_e_ln_b'], EBLK)
    g = global_features

    for s in range(STEPS):
        pe = 'pe%d' % s
        w1 = p[pe + '_w1']
        pq = _pq(node, w1[LATENT:2 * LATENT], w1[2 * LATENT:])
        ps, qr = _sc_gather2(pq[0], pq[1], senders, receivers)
        edge, esum = _edge_step(edge, ps, qr, w1[:LATENT], p[pe + '_b1'],
                                p[pe + '_w2'], p[pe + '_b2'], p[pe + '_w3'],
                                p[pe + '_b3'], p[pe + '_ln_s'],
                                p[pe + '_ln_b'])
        aggs = _sc_scatter(edge, receivers)
        pn = 'pn%d' % s
        a1 = p[pn + '_w1']
        node, nsum = _node_step(node, aggs[0], aggs[1], g, a1[:LATENT],
                                a1[LATENT:2 * LATENT], a1[2 * LATENT:],
                                p[pn + '_b1'], p[pn + '_w2'], p[pn + '_b2'],
                                p[pn + '_w3'], p[pn + '_b3'],
                                p[pn + '_ln_s'], p[pn + '_ln_b'])
        pg = 'pg%d' % s
        g1 = p[pg + '_w1']
        g = _global_step(g, nsum, esum, g1[:LATENT], g1[LATENT:2 * LATENT],
                         g1[2 * LATENT:], p[pg + '_b1'], p[pg + '_w2'],
                         p[pg + '_b2'], p[pg + '_w3'], p[pg + '_b3'],
                         p[pg + '_ln_s'], p[pg + '_ln_b'])

    return _decoder(node, p['dec_w1'], p['dec_b1'], p['dec_w2'], p['dec_b2'],
                    p['dec_w3'], p['dec_b3'])


def _foldb(mean, std, w1):
    return (mean / std) @ w1
